# Initial kernel scaffold; baseline (speedup 1.0000x reference)
import jax
import jax.numpy as jnp
from jax.experimental import pallas as pl


def kernel(input):
    raise NotImplementedError("write your pallas kernel here")


# speedup vs baseline: 1.1753x; 1.1753x over previous
import functools

import jax
import jax.numpy as jnp
from jax import lax
from jax.experimental import pallas as pl
from jax.experimental.pallas import tpu as pltpu
from jax.experimental.pallas import tpu_sc as plsc

F = 256
TRIL = F * (F + 1) // 2
OUT = F * F
DIAG_OFFSET = 0.5
NC = 2
NS = 16
NW = NC * NS


def _sc_body(in_hbm, out_hbm, outb, inb):
    cid = lax.axis_index("c")
    sid = lax.axis_index("s")
    wid = sid * NC + cid
    batch = in_hbm.shape[0]
    per_w = batch // NW
    base = wid * per_w

    iota16 = lax.iota(jnp.int32, 16)
    zeros16 = jnp.zeros((16,), jnp.float32)

    def zero_chunk(k, _):
        outb[pl.ds(k * 16, 16)] = zeros16
        return 0
    lax.fori_loop(0, OUT // 16, zero_chunk, 0)

    def batch_body(t, _):
        b = base + t
        pltpu.sync_copy(in_hbm.at[b], inb)

        def row_body(r, _):
            s = (r * (r + 1)) // 2
            jd = r // 16
            dst0 = r * F

            def copy_chunk(j, _):
                vals = inb[pl.ds(s + j * 16, 16)]
                outb[pl.ds(dst0 + j * 16, 16)] = vals
                return 0
            lax.fori_loop(0, jd, copy_chunk, 0)

            c = jd * 16 + iota16
            vals = inb[pl.ds(s + jd * 16, 16)]
            dval = jnp.abs(DIAG_OFFSET + vals) + 1e-9
            res = jnp.where(c < r, vals, jnp.where(c == r, dval, zeros16))
            outb[pl.ds(dst0 + jd * 16, 16)] = res
            return 0
        lax.fori_loop(0, F, row_body, 0)

        pltpu.sync_copy(outb, out_hbm.at[b])
        return 0
    lax.fori_loop(0, per_w, batch_body, 0)


def kernel(input):
    batch = input.shape[0]
    mesh = plsc.VectorSubcoreMesh(core_axis_name="c", subcore_axis_name="s")
    run = functools.partial(
        pl.kernel,
        mesh=mesh,
        out_type=jax.ShapeDtypeStruct((batch, OUT), jnp.float32),
        scratch_types=[
            pltpu.VMEM((OUT,), jnp.float32),
            pltpu.VMEM((TRIL,), jnp.float32),
        ],
    )(_sc_body)
    flat = run(input)
    return flat.reshape(batch, F, F)


# speedup vs baseline: 2.0365x; 1.7327x over previous
import functools

import jax
import jax.numpy as jnp
from jax import lax
from jax.experimental import pallas as pl
from jax.experimental.pallas import tpu as pltpu
from jax.experimental.pallas import tpu_sc as plsc

F = 256
TRIL = F * (F + 1) // 2
OUT = F * F
DIAG_OFFSET = 0.5
NC = 2
NS = 16
NW = NC * NS


def _sc_body(in_hbm, out_hbm, outb, inb):
    cid = lax.axis_index("c")
    sid = lax.axis_index("s")
    wid = sid * NC + cid
    batch = in_hbm.shape[0]
    per_w = batch // NW
    base = wid * per_w

    iota16 = lax.iota(jnp.int32, 16)
    zeros16 = jnp.zeros((16,), jnp.float32)

    @plsc.parallel_loop(0, OUT // 16, unroll=8)
    def _zero(k):
        outb[pl.ds(k * 16, 16)] = zeros16

    def batch_body(t, _):
        b = base + t
        pltpu.sync_copy(in_hbm.at[b], inb)

        for j in range(F // 16):
            col = j * 16

            @plsc.parallel_loop(j * 16 + 16, F, unroll=8)
            def _copy(r):
                s = (r * (r + 1)) >> 1
                outb[pl.ds(r * F + col, 16)] = inb[pl.ds(s + col, 16)]

        @plsc.parallel_loop(0, F, unroll=4)
        def _diag(r):
            s = (r * (r + 1)) >> 1
            jd16 = (r >> 4) * 16
            c = jd16 + iota16
            vals = inb[pl.ds(s + jd16, 16)]
            dval = jnp.abs(DIAG_OFFSET + vals) + 1e-9
            res = jnp.where(c < r, vals, jnp.where(c == r, dval, zeros16))
            outb[pl.ds(r * F + jd16, 16)] = res

        pltpu.sync_copy(outb, out_hbm.at[b])
        return 0
    lax.fori_loop(0, per_w, batch_body, 0)


def kernel(input):
    batch = input.shape[0]
    mesh = plsc.VectorSubcoreMesh(core_axis_name="c", subcore_axis_name="s")
    run = functools.partial(
        pl.kernel,
        mesh=mesh,
        out_type=jax.ShapeDtypeStruct((batch, OUT), jnp.float32),
        scratch_types=[
            pltpu.VMEM((OUT,), jnp.float32),
            pltpu.VMEM((TRIL,), jnp.float32),
        ],
    )(_sc_body)
    flat = run(input)
    return flat.reshape(batch, F, F)
